# baseline (speedup 1.0000x reference)
import functools

import jax
import jax.numpy as jnp
from jax import lax
from jax.experimental import pallas as pl
from jax.experimental.pallas import tpu as pltpu
from jax.experimental.pallas import tpu_sc as plsc

_SEQ = 8192
_DIM = 1024
_TOTAL = _SEQ * _DIM

_info = plsc.get_sparse_core_info()
_NC, _NS = _info.num_cores, _info.num_subcores
_NW = _NC * _NS
_WORDS_PER_W = _TOTAL // _NW

_CH = 32 * 1024
_NCHUNK = _WORDS_PER_W // _CH
_NB = 3

_mesh = plsc.VectorSubcoreMesh(core_axis_name="c", subcore_axis_name="s")


@functools.partial(
    pl.kernel,
    mesh=_mesh,
    out_type=jax.ShapeDtypeStruct((_TOTAL,), jnp.float32),
    scratch_types=(
        [pltpu.VMEM((_CH,), jnp.float32) for _ in range(_NB)]
        + [pltpu.SemaphoreType.DMA for _ in range(_NB)]
        + [pltpu.SemaphoreType.DMA for _ in range(_NB)]
    ),
)
def _pos_emb_copy(table_hbm, out_hbm, *scratch):
    bufs = scratch[:_NB]
    rsems = scratch[_NB:2 * _NB]
    wsems = scratch[2 * _NB:]

    wid = lax.axis_index("s") * _NC + lax.axis_index("c")
    base = wid * _WORDS_PER_W

    reads = [None] * _NCHUNK
    writes = [None] * _NCHUNK

    for i in range(min(_NB, _NCHUNK)):
        reads[i] = pltpu.async_copy(
            table_hbm.at[pl.ds(base + i * _CH, _CH)], bufs[i], rsems[i]
        )
    for i in range(_NCHUNK):
        b = i % _NB
        reads[i].wait()
        writes[i] = pltpu.async_copy(
            bufs[b], out_hbm.at[pl.ds(base + i * _CH, _CH)], wsems[b]
        )
        j = i + _NB
        if j < _NCHUNK:
            writes[i].wait()
            reads[j] = pltpu.async_copy(
                table_hbm.at[pl.ds(base + j * _CH, _CH)], bufs[b], rsems[b]
            )
    for i in range(max(0, _NCHUNK - _NB), _NCHUNK):
        writes[i].wait()


def kernel(hidden_embs, position_embedding_table):
    del hidden_embs
    flat = position_embedding_table.reshape(-1)
    return _pos_emb_copy(flat).reshape(_SEQ, _DIM)


# speedup vs baseline: 2.4443x; 2.4443x over previous
import functools

import jax
import jax.numpy as jnp
from jax import lax
from jax.experimental import pallas as pl
from jax.experimental.pallas import tpu as pltpu
from jax.experimental.pallas import tpu_sc as plsc

_SEQ = 8192
_DIM = 1024

_info = plsc.get_sparse_core_info()
_NC, _NS = _info.num_cores, _info.num_subcores
_NW = _NC * _NS
_ROWS_PER_W = _SEQ // _NW

_CH = 64
_NCHUNK = _ROWS_PER_W // _CH
_NB = 2

_mesh = plsc.VectorSubcoreMesh(core_axis_name="c", subcore_axis_name="s")


@functools.partial(
    pl.kernel,
    mesh=_mesh,
    out_type=jax.ShapeDtypeStruct((_SEQ, _DIM), jnp.float32),
    scratch_types=(
        [pltpu.VMEM_SHARED((_NS, _CH, _DIM), jnp.float32) for _ in range(_NB)]
        + [pltpu.SemaphoreType.DMA for _ in range(_NB)]
        + [pltpu.SemaphoreType.DMA for _ in range(_NB)]
    ),
)
def _pos_emb_copy(table_hbm, out_hbm, *scratch):
    bufs = scratch[:_NB]
    rsems = scratch[_NB:2 * _NB]
    wsems = scratch[2 * _NB:]

    sid = lax.axis_index("s")
    wid = sid * _NC + lax.axis_index("c")
    base = wid * _ROWS_PER_W

    reads = [None] * _NCHUNK
    writes = [None] * _NCHUNK
    for i in range(min(_NB, _NCHUNK)):
        reads[i] = pltpu.async_copy(
            table_hbm.at[pl.ds(base + i * _CH, _CH)], bufs[i].at[sid], rsems[i]
        )
    for i in range(_NCHUNK):
        b = i % _NB
        reads[i].wait()
        writes[i] = pltpu.async_copy(
            bufs[b].at[sid], out_hbm.at[pl.ds(base + i * _CH, _CH)], wsems[b]
        )
        j = i + _NB
        if j < _NCHUNK:
            writes[i].wait()
            reads[j] = pltpu.async_copy(
                table_hbm.at[pl.ds(base + j * _CH, _CH)], bufs[b].at[sid], rsems[b]
            )
    for i in range(max(0, _NCHUNK - _NB), _NCHUNK):
        writes[i].wait()


def kernel(hidden_embs, position_embedding_table):
    del hidden_embs
    return _pos_emb_copy(position_embedding_table)


# speedup vs baseline: 2.4828x; 1.0157x over previous
import functools

import jax
import jax.numpy as jnp
from jax import lax
from jax.experimental import pallas as pl
from jax.experimental.pallas import tpu as pltpu
from jax.experimental.pallas import tpu_sc as plsc

_SEQ = 8192
_DIM = 1024

_info = plsc.get_sparse_core_info()
_NC, _NS = _info.num_cores, _info.num_subcores
_NW = _NC * _NS
_ROWS_PER_W = _SEQ // _NW

_CH = 32
_NCHUNK = _ROWS_PER_W // _CH
_NB = 2

_mesh = plsc.VectorSubcoreMesh(core_axis_name="c", subcore_axis_name="s")


@functools.partial(
    pl.kernel,
    mesh=_mesh,
    out_type=jax.ShapeDtypeStruct((_SEQ, _DIM), jnp.float32),
    scratch_types=(
        [pltpu.VMEM((_CH, _DIM), jnp.float32) for _ in range(_NB)]
        + [pltpu.VMEM_SHARED((_NS, _CH, _DIM), jnp.float32) for _ in range(_NB)]
        + [pltpu.SemaphoreType.DMA for _ in range(4 * _NB)]
    ),
)
def _pos_emb_copy(table_hbm, out_hbm, *scratch):
    tbufs = scratch[:_NB]
    sbufs = scratch[_NB:2 * _NB]
    sems = scratch[2 * _NB:]
    rsems, wsems = sems[:2 * _NB], sems[2 * _NB:]

    sid = lax.axis_index("s")
    wid = sid * _NC + lax.axis_index("c")
    base = wid * _ROWS_PER_W

    def buf(i):
        path, slot = i % 2, (i // 2) % _NB
        if path == 0:
            return tbufs[slot]
        return sbufs[slot].at[sid]

    def ring(i):
        return (i % 2) * _NB + (i // 2) % _NB

    reads = [None] * _NCHUNK
    writes = [None] * _NCHUNK
    depth = 2 * _NB
    for i in range(min(depth, _NCHUNK)):
        reads[i] = pltpu.async_copy(
            table_hbm.at[pl.ds(base + i * _CH, _CH)], buf(i), rsems[ring(i)]
        )
    for i in range(_NCHUNK):
        reads[i].wait()
        writes[i] = pltpu.async_copy(
            buf(i), out_hbm.at[pl.ds(base + i * _CH, _CH)], wsems[ring(i)]
        )
        j = i + depth
        if j < _NCHUNK:
            writes[i].wait()
            reads[j] = pltpu.async_copy(
                table_hbm.at[pl.ds(base + j * _CH, _CH)], buf(j), rsems[ring(j)]
            )
    for i in range(max(0, _NCHUNK - depth), _NCHUNK):
        writes[i].wait()


def kernel(hidden_embs, position_embedding_table):
    del hidden_embs
    return _pos_emb_copy(position_embedding_table)


# speedup vs baseline: 2.5143x; 1.0127x over previous
import functools

import jax
import jax.numpy as jnp
from jax import lax
from jax.experimental import pallas as pl
from jax.experimental.pallas import tpu as pltpu
from jax.experimental.pallas import tpu_sc as plsc

_SEQ = 8192
_DIM = 1024

_info = plsc.get_sparse_core_info()
_NC, _NS = _info.num_cores, _info.num_subcores
_NW = _NC * _NS
_ROWS_PER_W = _SEQ // _NW

_CH = 32
_NCHUNK = _ROWS_PER_W // _CH
_NB = 3

_mesh = plsc.VectorSubcoreMesh(core_axis_name="c", subcore_axis_name="s")


@functools.partial(
    pl.kernel,
    mesh=_mesh,
    out_type=jax.ShapeDtypeStruct((_SEQ, _DIM), jnp.float32),
    scratch_types=(
        [pltpu.VMEM((_CH, _DIM), jnp.float32) for _ in range(_NB)]
        + [pltpu.SemaphoreType.DMA for _ in range(_NB)]
        + [pltpu.SemaphoreType.DMA for _ in range(_NB)]
    ),
)
def _pos_emb_copy(table_hbm, out_hbm, *scratch):
    bufs = scratch[:_NB]
    rsems = scratch[_NB:2 * _NB]
    wsems = scratch[2 * _NB:]

    wid = lax.axis_index("s") * _NC + lax.axis_index("c")
    base = wid * _ROWS_PER_W

    reads = [None] * _NCHUNK
    writes = [None] * _NCHUNK

    for i in range(min(_NB, _NCHUNK)):
        reads[i] = pltpu.async_copy(
            table_hbm.at[pl.ds(base + i * _CH, _CH)], bufs[i], rsems[i]
        )
    for i in range(_NCHUNK):
        b = i % _NB
        reads[i].wait()
        writes[i] = pltpu.async_copy(
            bufs[b], out_hbm.at[pl.ds(base + i * _CH, _CH)], wsems[b]
        )
        j = i + _NB
        if j < _NCHUNK:
            writes[i].wait()
            reads[j] = pltpu.async_copy(
                table_hbm.at[pl.ds(base + j * _CH, _CH)], bufs[b], rsems[b]
            )
    for i in range(max(0, _NCHUNK - _NB), _NCHUNK):
        writes[i].wait()


def kernel(hidden_embs, position_embedding_table):
    del hidden_embs
    return _pos_emb_copy(position_embedding_table)
